# Initial kernel scaffold; baseline (speedup 1.0000x reference)
import jax
import jax.numpy as jnp
from jax.experimental import pallas as pl


def kernel(x, W_gate, b_gate, W1, b1, W2, b2):
    raise NotImplementedError("write your pallas kernel here")


# speedup vs baseline: 1.2190x; 1.2190x over previous
import functools

import jax
import jax.numpy as jnp
from jax import lax
from jax.experimental import pallas as pl
from jax.experimental.pallas import tpu as pltpu
from jax.experimental.pallas import tpu_sc as plsc

T = 2048
D = 768
H = 768
E = 8
BLK = 128
NB = T // BLK + E
P = NB * BLK
LANES = 128

_NC, _NS = 2, 16
_NW = _NC * _NS
_RPW = T // _NW


def _router_body(x_ref, wg_ref, bg_ref, pos_ref, w_ref, meta_ref):
    x = x_ref[...]
    logits = jnp.dot(x, wg_ref[...], preferred_element_type=jnp.float32)
    logits = logits + bg_ref[...]
    col = lax.broadcasted_iota(jnp.int32, (T, LANES), 1)
    valid = col < E
    lm = jnp.where(valid, logits, -1e30)
    m = jnp.max(lm, axis=1, keepdims=True)
    ex = jnp.where(valid, jnp.exp(lm - m), 0.0)
    s = jnp.sum(ex, axis=1, keepdims=True)
    w_ref[...] = 1.0 / s
    idx = jnp.min(jnp.where((lm == m) & valid, col, E), axis=1, keepdims=True)
    oh = (col == idx) & valid
    ohi = oh.astype(jnp.int32)

    c = ohi
    sh = 1
    while sh < T:
        c = c + jnp.concatenate(
            [jnp.zeros((sh, LANES), jnp.int32), c[: T - sh, :]], axis=0)
        sh *= 2
    counts = c[T - 1 : T, :]
    rank = jnp.sum(c * ohi, axis=1, keepdims=True) - 1

    nb = (counts + (BLK - 1)) // BLK
    cb = nb
    sh = 1
    while sh < E:
        cb = cb + jnp.concatenate(
            [jnp.zeros((1, sh), jnp.int32), cb[:, : LANES - sh]], axis=1)
        sh *= 2
    start_blk = cb - nb
    start_row = start_blk * BLK
    pos_ref[...] = (
        jnp.sum(jnp.where(oh, jnp.broadcast_to(start_row, (T, LANES)), 0),
                axis=1, keepdims=True) + rank)

    used = cb[0:1, E - 1 : E]
    i_ids = lax.broadcasted_iota(jnp.int32, (32, LANES), 0)
    i_eff = jnp.minimum(i_ids, jnp.broadcast_to(used, (32, LANES)) - 1)
    sb = jnp.broadcast_to(start_blk, (32, LANES))
    vale = lax.broadcasted_iota(jnp.int32, (32, LANES), 1) < E
    be = jnp.sum(jnp.where(vale & (sb <= i_eff), 1, 0), axis=1, keepdims=True) - 1
    r_ids = lax.broadcasted_iota(jnp.int32, (32, 1), 0)
    meta_ref[...] = jnp.where(r_ids == NB, jnp.broadcast_to(used, (32, 1)), be)


def _router(x2d, wg_pad, bg_pad):
    return pl.pallas_call(
        _router_body,
        out_shape=(
            jax.ShapeDtypeStruct((T, 1), jnp.int32),
            jax.ShapeDtypeStruct((T, 1), jnp.float32),
            jax.ShapeDtypeStruct((32, 1), jnp.int32),
        ),
    )(x2d, wg_pad, bg_pad)


@functools.cache
def _sc_mesh():
    return plsc.VectorSubcoreMesh(
        core_axis_name="c", subcore_axis_name="s",
        num_cores=_NC, num_subcores=_NS)


def _sc_scatter_body(x_hbm, pos_hbm, out_hbm, idx_v, rows_v, sem):
    wid = lax.axis_index("s") * _NC + lax.axis_index("c")
    base = wid * _RPW
    pltpu.sync_copy(pos_hbm.at[pl.ds(base, _RPW)], idx_v)
    pltpu.sync_copy(x_hbm.at[pl.ds(base, _RPW)], rows_v)
    pltpu.async_copy(rows_v, out_hbm.at[idx_v], sem).wait()


def _sc_scatter(x2d, pos):
    return pl.kernel(
        _sc_scatter_body,
        out_type=jax.ShapeDtypeStruct((P, D), jnp.float32),
        mesh=_sc_mesh(),
        scratch_types=[
            pltpu.VMEM((_RPW,), jnp.int32),
            pltpu.VMEM((_RPW, D), jnp.float32),
            pltpu.SemaphoreType.DMA,
        ],
    )(x2d, pos)


def _sc_gather_body(ys_hbm, pos_hbm, out_hbm, idx_v, rows_v, sem):
    wid = lax.axis_index("s") * _NC + lax.axis_index("c")
    base = wid * _RPW
    pltpu.sync_copy(pos_hbm.at[pl.ds(base, _RPW)], idx_v)
    pltpu.async_copy(ys_hbm.at[idx_v], rows_v, sem).wait()
    pltpu.sync_copy(rows_v, out_hbm.at[pl.ds(base, _RPW)])


def _sc_gather(y_sorted, pos):
    return pl.kernel(
        _sc_gather_body,
        out_type=jax.ShapeDtypeStruct((T, D), jnp.float32),
        mesh=_sc_mesh(),
        scratch_types=[
            pltpu.VMEM((_RPW,), jnp.int32),
            pltpu.VMEM((_RPW, D), jnp.float32),
            pltpu.SemaphoreType.DMA,
        ],
    )(y_sorted, pos)


def _ffn_body(be_ref, used_ref, x_ref, w1_ref, b1_ref, w2_ref, b2_ref, y_ref):
    i = pl.program_id(0)

    @pl.when(i < used_ref[0])
    def _():
        xb = x_ref[...]
        h = jnp.dot(xb, w1_ref[0], preferred_element_type=jnp.float32)
        h = h + b1_ref[0]
        h = h * (1.0 / (1.0 + jnp.exp(-h)))
        y = jnp.dot(h, w2_ref[0], preferred_element_type=jnp.float32)
        y_ref[...] = y + b2_ref[0]


def _ffn(be, used, x_sorted, W1, b1, W2, b2):
    grid_spec = pltpu.PrefetchScalarGridSpec(
        num_scalar_prefetch=2,
        grid=(NB,),
        in_specs=[
            pl.BlockSpec((BLK, D), lambda i, be, ub: (jnp.minimum(i, ub[0] - 1), 0)),
            pl.BlockSpec((1, D, H), lambda i, be, ub: (be[i], 0, 0)),
            pl.BlockSpec((1, 1, H), lambda i, be, ub: (be[i], 0, 0)),
            pl.BlockSpec((1, H, D), lambda i, be, ub: (be[i], 0, 0)),
            pl.BlockSpec((1, 1, D), lambda i, be, ub: (be[i], 0, 0)),
        ],
        out_specs=pl.BlockSpec((BLK, D), lambda i, be, ub: (i, 0)),
    )
    return pl.pallas_call(
        _ffn_body,
        grid_spec=grid_spec,
        out_shape=jax.ShapeDtypeStruct((P, D), jnp.float32),
    )(be, used, x_sorted, W1, b1.reshape(E, 1, H), W2, b2.reshape(E, 1, D))


def _scale_body(y_ref, w_ref, o_ref):
    o_ref[...] = y_ref[...] * w_ref[...]


def _scale(y2d, w):
    return pl.pallas_call(
        _scale_body,
        grid=(8,),
        in_specs=[
            pl.BlockSpec((T // 8, D), lambda i: (i, 0)),
            pl.BlockSpec((T // 8, 1), lambda i: (i, 0)),
        ],
        out_specs=pl.BlockSpec((T // 8, D), lambda i: (i, 0)),
        out_shape=jax.ShapeDtypeStruct((T, D), jnp.float32),
    )(y2d, w)


@jax.jit
def kernel(x, W_gate, b_gate, W1, b1, W2, b2):
    x2d = x.reshape(T, D)
    wg_pad = jnp.zeros((D, LANES), jnp.float32).at[:, :E].set(W_gate)
    bg_pad = jnp.zeros((1, LANES), jnp.float32).at[0, :E].set(b_gate)

    pos, w, meta = _router(x2d, wg_pad, bg_pad)
    pos_flat = pos.reshape(T)
    be = meta[:NB, 0]
    used = meta[NB : NB + 1, 0]

    x_sorted = _sc_scatter(x2d, pos_flat)
    y_sorted = _ffn(be, used, x_sorted, W1, b1, W2, b2)
    y2d = _sc_gather(y_sorted, pos_flat)
    y = _scale(y2d, w)
    return y.reshape(x.shape)


# speedup vs baseline: 1.4134x; 1.1595x over previous
import functools

import jax
import jax.numpy as jnp
from jax import lax
from jax.experimental import pallas as pl
from jax.experimental.pallas import tpu as pltpu
from jax.experimental.pallas import tpu_sc as plsc

T = 2048
D = 768
H = 768
E = 8
DA = 896
BLK = 256
NB = T // BLK + E
P = NB * BLK
LANES = 128

_NC, _NS = 2, 16
_NW = _NC * _NS
_RPW = T // _NW


def _router_body(x_ref, wg_ref, bg_ref, pos_ref, meta_ref, wcol_ref):
    x = x_ref[...]
    logits = jnp.dot(x, wg_ref[...], preferred_element_type=jnp.float32)
    logits = logits + bg_ref[...]
    col = lax.broadcasted_iota(jnp.int32, (T, LANES), 1)
    valid = col < E
    lm = jnp.where(valid, logits, -1e30)
    m = jnp.max(lm, axis=1, keepdims=True)
    ex = jnp.where(valid, jnp.exp(lm - m), 0.0)
    s = jnp.sum(ex, axis=1, keepdims=True)
    w = 1.0 / s
    idx = jnp.min(jnp.where((lm == m) & valid, col, E), axis=1, keepdims=True)
    oh = (col == idx) & valid
    ohi = oh.astype(jnp.int32)

    wcol_ref[...] = jnp.where(col[:, : DA - D] == 0, w, 0.0)

    c = ohi
    sh = 1
    while sh < T:
        c = c + jnp.concatenate(
            [jnp.zeros((sh, LANES), jnp.int32), c[: T - sh, :]], axis=0)
        sh *= 2
    counts = c[T - 1 : T, :]
    rank = jnp.sum(c * ohi, axis=1, keepdims=True) - 1

    nb = (counts + (BLK - 1)) // BLK
    cb = nb
    sh = 1
    while sh < E:
        cb = cb + jnp.concatenate(
            [jnp.zeros((1, sh), jnp.int32), cb[:, : LANES - sh]], axis=1)
        sh *= 2
    start_blk = cb - nb
    start_row = start_blk * BLK
    pos_ref[...] = (
        jnp.sum(jnp.where(oh, jnp.broadcast_to(start_row, (T, LANES)), 0),
                axis=1, keepdims=True) + rank)

    used = cb[0:1, E - 1 : E]
    i_ids = lax.broadcasted_iota(jnp.int32, (32, LANES), 0)
    i_eff = jnp.minimum(i_ids, jnp.broadcast_to(used, (32, LANES)) - 1)
    sb = jnp.broadcast_to(start_blk, (32, LANES))
    vale = lax.broadcasted_iota(jnp.int32, (32, LANES), 1) < E
    be = jnp.sum(jnp.where(vale & (sb <= i_eff), 1, 0), axis=1, keepdims=True) - 1
    r_ids = lax.broadcasted_iota(jnp.int32, (32, 1), 0)
    meta_ref[...] = jnp.where(r_ids == NB, jnp.broadcast_to(used, (32, 1)), be)


def _router(x2d, wg_pad, bg_pad):
    return pl.pallas_call(
        _router_body,
        out_shape=(
            jax.ShapeDtypeStruct((T, 1), jnp.int32),
            jax.ShapeDtypeStruct((32, 1), jnp.int32),
            jax.ShapeDtypeStruct((T, DA - D), jnp.float32),
        ),
    )(x2d, wg_pad, bg_pad)


@functools.cache
def _sc_mesh():
    return plsc.VectorSubcoreMesh(
        core_axis_name="c", subcore_axis_name="s",
        num_cores=_NC, num_subcores=_NS)


def _sc_scatter_body(x_hbm, wcol_hbm, pos_hbm, out_hbm, idx_v, rows_v, sem):
    wid = lax.axis_index("s") * _NC + lax.axis_index("c")
    base = wid * _RPW
    pltpu.sync_copy(pos_hbm.at[pl.ds(base, _RPW)], idx_v)
    pltpu.sync_copy(x_hbm.at[pl.ds(base, _RPW)], rows_v.at[:, pl.ds(0, D)])
    pltpu.sync_copy(wcol_hbm.at[pl.ds(base, _RPW)], rows_v.at[:, pl.ds(D, DA - D)])
    pltpu.async_copy(rows_v, out_hbm.at[idx_v], sem).wait()


def _sc_scatter(x2d, wcol, pos):
    return pl.kernel(
        _sc_scatter_body,
        out_type=jax.ShapeDtypeStruct((P, DA), jnp.float32),
        mesh=_sc_mesh(),
        scratch_types=[
            pltpu.VMEM((_RPW,), jnp.int32),
            pltpu.VMEM((_RPW, DA), jnp.float32),
            pltpu.SemaphoreType.DMA,
        ],
    )(x2d, wcol, pos)


def _sc_gather_body(ys_hbm, pos_hbm, out_hbm, idx_v, rows_v, sem):
    wid = lax.axis_index("s") * _NC + lax.axis_index("c")
    base = wid * _RPW
    pltpu.sync_copy(pos_hbm.at[pl.ds(base, _RPW)], idx_v)
    pltpu.async_copy(ys_hbm.at[idx_v], rows_v, sem).wait()
    pltpu.sync_copy(rows_v, out_hbm.at[pl.ds(base, _RPW)])


def _sc_gather(y_sorted, pos):
    return pl.kernel(
        _sc_gather_body,
        out_type=jax.ShapeDtypeStruct((T, D), jnp.float32),
        mesh=_sc_mesh(),
        scratch_types=[
            pltpu.VMEM((_RPW,), jnp.int32),
            pltpu.VMEM((_RPW, D), jnp.float32),
            pltpu.SemaphoreType.DMA,
        ],
    )(y_sorted, pos)


def _ffn_body(m_ref, x_ref, w1_ref, b1_ref, w2_ref, b2_ref, y_ref):
    i = pl.program_id(0)

    @pl.when(i < m_ref[NB])
    def _():
        xa = x_ref[...]
        xb = xa[:, :D].astype(jnp.bfloat16)
        w1 = w1_ref[0].astype(jnp.bfloat16)
        h = jnp.dot(xb, w1, preferred_element_type=jnp.float32)
        h = h + b1_ref[0]
        h = h * (1.0 / (1.0 + jnp.exp(-h)))
        w2 = w2_ref[0].astype(jnp.bfloat16)
        y = jnp.dot(h.astype(jnp.bfloat16), w2, preferred_element_type=jnp.float32)
        y_ref[...] = (y + b2_ref[0]) * xa[:, D : D + 1]


def _ffn(meta, x_sorted, W1, b1, W2, b2):
    grid_spec = pltpu.PrefetchScalarGridSpec(
        num_scalar_prefetch=1,
        grid=(NB,),
        in_specs=[
            pl.BlockSpec((BLK, DA), lambda i, m: (jnp.minimum(i, m[NB] - 1), 0)),
            pl.BlockSpec((1, D, H), lambda i, m: (m[i], 0, 0)),
            pl.BlockSpec((1, 1, H), lambda i, m: (m[i], 0, 0)),
            pl.BlockSpec((1, H, D), lambda i, m: (m[i], 0, 0)),
            pl.BlockSpec((1, 1, D), lambda i, m: (m[i], 0, 0)),
        ],
        out_specs=pl.BlockSpec((BLK, D), lambda i, m: (jnp.minimum(i, m[NB] - 1), 0)),
    )
    return pl.pallas_call(
        _ffn_body,
        grid_spec=grid_spec,
        out_shape=jax.ShapeDtypeStruct((P, D), jnp.float32),
    )(meta, x_sorted, W1, b1.reshape(E, 1, H), W2, b2.reshape(E, 1, D))


@jax.jit
def kernel(x, W_gate, b_gate, W1, b1, W2, b2):
    x2d = x.reshape(T, D)
    wg_pad = jnp.zeros((D, LANES), jnp.float32).at[:, :E].set(W_gate)
    bg_pad = jnp.zeros((1, LANES), jnp.float32).at[0, :E].set(b_gate)

    pos, meta, wcol = _router(x2d, wg_pad, bg_pad)
    pos_flat = pos.reshape(T)
    meta_flat = meta.reshape(32)

    x_sorted = _sc_scatter(x2d, wcol, pos_flat)
    y_sorted = _ffn(meta_flat, x_sorted, W1, b1, W2, b2)
    y2d = _sc_gather(y_sorted, pos_flat)
    return y2d.reshape(x.shape)


# speedup vs baseline: 1.5727x; 1.1127x over previous
import functools

import jax
import jax.numpy as jnp
from jax import lax
from jax.experimental import pallas as pl
from jax.experimental.pallas import tpu as pltpu
from jax.experimental.pallas import tpu_sc as plsc

T = 2048
D = 768
H = 768
E = 8
DA = 896
BLK = 256
NB = T // BLK + E
P = NB * BLK
LANES = 128

_NC, _NS = 2, 16
_NW = _NC * _NS
_RPW = T // _NW


def _router_body(x_ref, wg_ref, bg_ref, pos_ref, meta_ref, wcol_ref):
    x = x_ref[...]
    logits = jnp.dot(x, wg_ref[...], preferred_element_type=jnp.float32)
    logits = logits + jnp.reshape(bg_ref[...], (1, E))
    col = lax.broadcasted_iota(jnp.int32, (T, E), 1)
    m = jnp.max(logits, axis=1, keepdims=True)
    ex = jnp.exp(logits - m)
    s = jnp.sum(ex, axis=1, keepdims=True)
    w = 1.0 / s
    idx = jnp.min(jnp.where(logits == m, col, E), axis=1, keepdims=True)
    oh = col == idx
    ohi = oh.astype(jnp.int32)

    c128 = lax.broadcasted_iota(jnp.int32, (T, DA - D), 1)
    wcol_ref[...] = jnp.where(c128 == 0, w, 0.0)

    c = ohi
    sh = 1
    while sh < T:
        c = c + jnp.concatenate(
            [jnp.zeros((sh, E), jnp.int32), c[: T - sh, :]], axis=0)
        sh *= 2
    counts = c[T - 1 : T, :]
    rank = jnp.sum(c * ohi, axis=1, keepdims=True) - 1

    nb = (counts + (BLK - 1)) // BLK
    cb = nb
    sh = 1
    while sh < E:
        cb = cb + jnp.concatenate(
            [jnp.zeros((1, sh), jnp.int32), cb[:, : E - sh]], axis=1)
        sh *= 2
    start_blk = cb - nb
    start_row = start_blk * BLK
    pos_ref[...] = (
        jnp.sum(jnp.where(oh, jnp.broadcast_to(start_row, (T, E)), 0),
                axis=1, keepdims=True) + rank)

    used = cb[0:1, E - 1 : E]
    i_ids = lax.broadcasted_iota(jnp.int32, (32, E), 0)
    i_eff = jnp.minimum(i_ids, jnp.broadcast_to(used, (32, E)) - 1)
    sb = jnp.broadcast_to(start_blk, (32, E))
    be = jnp.sum(jnp.where(sb <= i_eff, 1, 0), axis=1, keepdims=True) - 1
    r_ids = lax.broadcasted_iota(jnp.int32, (32, 1), 0)
    meta_ref[...] = jnp.where(r_ids == NB, jnp.broadcast_to(used, (32, 1)), be)


def _router(x2d, W_gate, b_gate):
    return pl.pallas_call(
        _router_body,
        out_shape=(
            jax.ShapeDtypeStruct((T, 1), jnp.int32),
            jax.ShapeDtypeStruct((32, 1), jnp.int32),
            jax.ShapeDtypeStruct((T, DA - D), jnp.float32),
        ),
    )(x2d, W_gate, b_gate)


@functools.cache
def _sc_mesh():
    return plsc.VectorSubcoreMesh(
        core_axis_name="c", subcore_axis_name="s",
        num_cores=_NC, num_subcores=_NS)


def _sc_scatter_body(x_hbm, wcol_hbm, pos_hbm, out_hbm, idx_v, rows_v, sem):
    wid = lax.axis_index("s") * _NC + lax.axis_index("c")
    base = wid * _RPW
    pltpu.sync_copy(pos_hbm.at[pl.ds(base, _RPW)], idx_v)
    pltpu.sync_copy(x_hbm.at[pl.ds(base, _RPW)], rows_v.at[:, pl.ds(0, D)])
    pltpu.sync_copy(wcol_hbm.at[pl.ds(base, _RPW)], rows_v.at[:, pl.ds(D, DA - D)])
    pltpu.async_copy(rows_v, out_hbm.at[idx_v], sem).wait()


def _sc_scatter(x2d, wcol, pos):
    return pl.kernel(
        _sc_scatter_body,
        out_type=jax.ShapeDtypeStruct((P, DA), jnp.float32),
        mesh=_sc_mesh(),
        scratch_types=[
            pltpu.VMEM((_RPW,), jnp.int32),
            pltpu.VMEM((_RPW, DA), jnp.float32),
            pltpu.SemaphoreType.DMA,
        ],
    )(x2d, wcol, pos)


def _sc_gather_body(ys_hbm, pos_hbm, out_hbm, idx_v, rows_v, sem):
    wid = lax.axis_index("s") * _NC + lax.axis_index("c")
    base = wid * _RPW
    pltpu.sync_copy(pos_hbm.at[pl.ds(base, _RPW)], idx_v)
    pltpu.async_copy(ys_hbm.at[idx_v], rows_v, sem).wait()
    pltpu.sync_copy(rows_v, out_hbm.at[pl.ds(base, _RPW)])


def _sc_gather(y_sorted, pos):
    return pl.kernel(
        _sc_gather_body,
        out_type=jax.ShapeDtypeStruct((T, D), jnp.float32),
        mesh=_sc_mesh(),
        scratch_types=[
            pltpu.VMEM((_RPW,), jnp.int32),
            pltpu.VMEM((_RPW, D), jnp.float32),
            pltpu.SemaphoreType.DMA,
        ],
    )(y_sorted, pos)


def _ffn_body(m_ref, x_ref, w1_ref, b1_ref, w2_ref, b2_ref, y_ref):
    i = pl.program_id(0)

    @pl.when(i < m_ref[NB])
    def _():
        xa = x_ref[...]
        xb = xa[:, :D].astype(jnp.bfloat16)
        w1 = w1_ref[0].astype(jnp.bfloat16)
        h = jnp.dot(xb, w1, preferred_element_type=jnp.float32)
        h = h + b1_ref[0]
        h = h * (1.0 / (1.0 + jnp.exp(-h)))
        w2 = w2_ref[0].astype(jnp.bfloat16)
        y = jnp.dot(h.astype(jnp.bfloat16), w2, preferred_element_type=jnp.float32)
        y_ref[...] = (y + b2_ref[0]) * xa[:, D : D + 1]


def _ffn(meta, x_sorted, W1, b1, W2, b2):
    grid_spec = pltpu.PrefetchScalarGridSpec(
        num_scalar_prefetch=1,
        grid=(NB,),
        in_specs=[
            pl.BlockSpec((BLK, DA), lambda i, m: (jnp.minimum(i, m[NB] - 1), 0)),
            pl.BlockSpec((1, D, H), lambda i, m: (m[i], 0, 0)),
            pl.BlockSpec((1, 1, H), lambda i, m: (m[i], 0, 0)),
            pl.BlockSpec((1, H, D), lambda i, m: (m[i], 0, 0)),
            pl.BlockSpec((1, 1, D), lambda i, m: (m[i], 0, 0)),
        ],
        out_specs=pl.BlockSpec((BLK, D), lambda i, m: (jnp.minimum(i, m[NB] - 1), 0)),
    )
    return pl.pallas_call(
        _ffn_body,
        grid_spec=grid_spec,
        out_shape=jax.ShapeDtypeStruct((P, D), jnp.float32),
    )(meta, x_sorted, W1, b1.reshape(E, 1, H), W2, b2.reshape(E, 1, D))


@jax.jit
def kernel(x, W_gate, b_gate, W1, b1, W2, b2):
    x2d = x.reshape(T, D)
    pos, meta, wcol = _router(x2d, W_gate, b_gate)
    pos_flat = pos.reshape(T)
    meta_flat = meta.reshape(32)

    x_sorted = _sc_scatter(x2d, wcol, pos_flat)
    y_sorted = _ffn(meta_flat, x_sorted, W1, b1, W2, b2)
    y2d = _sc_gather(y_sorted, pos_flat)
    return y2d.reshape(x.shape)
